# baseline (speedup 1.0000x reference)
import functools
import numpy as np
import jax
import jax.numpy as jnp
from jax import lax
from jax.experimental import pallas as pl
from jax.experimental.pallas import tpu as pltpu
from jax.experimental.pallas import tpu_sc as plsc

_IGNORE = 255
_MIN_KEPT = 65536
_B, _C, _H, _W = 4, 19, 512, 512
_N = _B * _H * _W
_ROWS = _H * _W // 128
_RBLK = 64

_NT = 16
_CHUNK = _N // _NT
_NV = _CHUNK // 16
_BINS = 256
_NREG = 8
_INT_MIN = np.int32(-2147483648)


def _key_i32(s):
    b = lax.bitcast_convert_type(s, jnp.int32)
    return b ^ (lax.shift_right_arithmetic(b, 31) | _INT_MIN)


def _thresh_const():
    b = np.float32(np.log(np.float32(0.6))).view(np.int32)
    u = np.uint32(b) ^ (np.uint32(np.int32(b) >> np.int32(31))
                        | np.uint32(0x80000000))
    return int(np.int32(u ^ np.uint32(0x80000000)))

_T06_SIGNED = _thresh_const()


def _tc_body(pred_ref, tgt_ref, key_ref):
    x = pred_ref[0]
    t = tgt_ref[0]
    m = jnp.max(x, axis=0)
    lse = m + jnp.log(jnp.sum(jnp.exp(x - m[None]), axis=0))
    cls = lax.broadcasted_iota(jnp.int32, x.shape, 0)
    xt = jnp.max(jnp.where(cls == t[None], x, -jnp.inf), axis=0)
    s = jnp.where(t != _IGNORE, xt - lse, jnp.float32(1e30))
    key_ref[0] = _key_i32(s).reshape(_RBLK * 4, 128)


_tc_keys = pl.pallas_call(
    _tc_body,
    grid=(_B, _H // _RBLK),
    in_specs=[
        pl.BlockSpec((1, _C, _RBLK, _W), lambda b, i: (b, 0, i, 0)),
        pl.BlockSpec((1, _RBLK, _W), lambda b, i: (b, i, 0)),
    ],
    out_specs=pl.BlockSpec((1, _RBLK * 4, 128), lambda b, i: (b, i, 0)),
    out_shape=jax.ShapeDtypeStruct((_B, _ROWS, 128), jnp.int32),
)


def _sc_body(keys_hbm, out_hbm, keys_v, hist_v, hist2_v, histr_v, ctrl_v,
             red_v, outst_v, hist_sh, ctrl_sh, sums_sh, cnts_sh):
    c = lax.axis_index("c")
    sid = lax.axis_index("s")
    lane = lax.iota(jnp.int32, 16)
    lane_base = lane * _BINS
    ones = jnp.ones((16,), jnp.int32)

    pltpu.sync_copy(keys_hbm.at[pl.ds(sid * _CHUNK, _CHUNK)], keys_v)

    @plsc.parallel_loop(0, _NREG * _NT * _BINS // 16, unroll=8)
    def _(i):
        hist_v[pl.ds(i * 16, 16)] = jnp.zeros((16,), jnp.int32)

    prefix = jnp.zeros((16,), jnp.int32)
    krem = jnp.full((16,), _MIN_KEPT, jnp.int32)

    for p in range(4):
        shift = jnp.full((16,), 24 - 8 * p, jnp.int32)
        hshift = jnp.full((16,), 32 - 8 * p, jnp.int32)
        mask8 = jnp.full((16,), 0xFF, jnp.int32)
        nreg = _NREG if p == 0 else 2

        @plsc.parallel_loop(0, _NV // 2, unroll=4)
        def _(i, prefix=prefix):
            for r in range(2):
                v = keys_v[pl.ds((2 * i + r) * 16, 16)]
                idx = lane_base + (lax.shift_right_logical(v, shift) & mask8)
                reg = (2 * i + r) % nreg if nreg > 2 else r
                idx = idx + reg * (_NT * _BINS)
                if p == 0:
                    plsc.addupdate_scatter(hist_v, [idx], ones)
                else:
                    cand = lax.shift_right_logical(v, hshift) == prefix
                    plsc.addupdate_scatter(hist_v, [idx], ones, mask=cand)

        @plsc.parallel_loop(0, _BINS // 16, unroll=2)
        def _(j):
            acc = hist_v[pl.ds(j * 16, 16)]
            hist_v[pl.ds(j * 16, 16)] = jnp.zeros((16,), jnp.int32)
            for l in range(1, nreg * _NT):
                acc = acc + hist_v[pl.ds(l * _BINS + j * 16, 16)]
                hist_v[pl.ds(l * _BINS + j * 16, 16)] = jnp.zeros((16,), jnp.int32)
            histr_v[pl.ds(j * 16, 16)] = acc

        pltpu.sync_copy(histr_v, hist_sh.at[pl.ds(sid * _BINS, _BINS)])
        plsc.subcore_barrier()

        @pl.when(sid == 0)
        def _():
            pltpu.sync_copy(hist_sh, hist2_v)
            carry = jnp.zeros((16,), jnp.int32)
            dsel = jnp.zeros((16,), jnp.int32)
            below = jnp.zeros((16,), jnp.int32)
            for j in range(_BINS // 16):
                tot = hist2_v[pl.ds(j * 16, 16)]
                for tt in range(1, _NT):
                    tot = tot + hist2_v[pl.ds(tt * _BINS + j * 16, 16)]
                cum = plsc.cumsum(tot) + carry
                prev = cum - tot
                isd = (cum >= krem) & (prev < krem)
                bid = jnp.full((16,), j * 16, jnp.int32) + lane
                dsel = dsel + jnp.where(isd, bid, 0)
                below = below + jnp.where(isd, prev, 0)
                carry = carry + jnp.broadcast_to(jnp.sum(tot), (16,))
            dvec = jnp.broadcast_to(jnp.sum(dsel), (16,))
            bvec = jnp.broadcast_to(jnp.sum(below), (16,))
            eight = jnp.full((16,), 8, jnp.int32)
            ctrl_v[pl.ds(0, 16)] = lax.shift_left(prefix, eight) | dvec
            ctrl_v[pl.ds(16, 16)] = krem - bvec
            pltpu.sync_copy(ctrl_v, ctrl_sh)

        plsc.subcore_barrier()
        pltpu.sync_copy(ctrl_sh, ctrl_v)
        prefix = ctrl_v[pl.ds(0, 16)]
        krem = ctrl_v[pl.ds(16, 16)]

    tsig = jnp.maximum(prefix ^ _INT_MIN,
                       jnp.full((16,), _T06_SIGNED, jnp.int32))
    start = c * (_NV // 2)

    zf = jnp.zeros((16,), jnp.float32)

    @plsc.parallel_loop(0, _NV // 4, unroll=4, carry=(zf, zf, zf, zf))
    def facc(i, acc):
        a0, c0, a1, c1 = acc
        for r in range(2):
            v = keys_v[pl.ds((start + 2 * i + r) * 16, 16)]
            kept = (v ^ _INT_MIN) <= tsig
            nv = v ^ jnp.full((16,), -1, jnp.int32)
            b = v ^ (lax.shift_right_arithmetic(nv, 31) | _INT_MIN)
            s = plsc.bitcast(b, jnp.float32)
            ds = jnp.where(kept, -s, jnp.float32(0.0))
            dc = jnp.where(kept, jnp.float32(1.0), jnp.float32(0.0))
            if r == 0:
                a0, c0 = a0 + ds, c0 + dc
            else:
                a1, c1 = a1 + ds, c1 + dc
        return (a0, c0, a1, c1)

    asum = facc[0] + facc[2]
    acnt = facc[1] + facc[3]

    outst_v[...] = asum
    pltpu.sync_copy(outst_v, sums_sh.at[pl.ds(sid * 16, 16)])
    outst_v[...] = acnt
    pltpu.sync_copy(outst_v, cnts_sh.at[pl.ds(sid * 16, 16)])
    plsc.subcore_barrier()

    @pl.when(sid == 0)
    def _():
        pltpu.sync_copy(sums_sh, red_v)
        stot = red_v[pl.ds(0, 16)]
        for tt in range(1, _NT):
            stot = stot + red_v[pl.ds(tt * 16, 16)]
        ssum = jnp.broadcast_to(jnp.sum(stot), (16,))
        pltpu.sync_copy(cnts_sh, red_v)
        ctot = red_v[pl.ds(0, 16)]
        for tt in range(1, _NT):
            ctot = ctot + red_v[pl.ds(tt * 16, 16)]
        csum = jnp.broadcast_to(jnp.sum(ctot), (16,))
        outst_v[...] = jnp.where(lane < 8, ssum, csum)
        pltpu.sync_copy(outst_v, out_hbm.at[c])


@functools.cache
def _get_sc_select():
    return functools.partial(
        pl.kernel,
        out_type=jax.ShapeDtypeStruct((2, 16), jnp.float32),
        mesh=plsc.VectorSubcoreMesh(core_axis_name="c", subcore_axis_name="s"),
        compiler_params=pltpu.CompilerParams(needs_layout_passes=False),
        scratch_types=[
            pltpu.VMEM((_CHUNK,), jnp.int32),
            pltpu.VMEM((_NREG * _NT * _BINS,), jnp.int32),
            pltpu.VMEM((_NT * _BINS,), jnp.int32),
            pltpu.VMEM((_BINS,), jnp.int32),
            pltpu.VMEM((32,), jnp.int32),
            pltpu.VMEM((_NT * 16,), jnp.float32),
            pltpu.VMEM((16,), jnp.float32),
            pltpu.VMEM_SHARED((_NT * _BINS,), jnp.int32),
            pltpu.VMEM_SHARED((32,), jnp.int32),
            pltpu.VMEM_SHARED((_NT * 16,), jnp.float32),
            pltpu.VMEM_SHARED((_NT * 16,), jnp.float32),
        ],
    )(_sc_body)


def kernel(pred, target):
    keys = _tc_keys(pred, target.astype(jnp.int32)).reshape(-1)
    out = _get_sc_select()(keys)
    tot = out[0, 0] + out[1, 0]
    cnt = out[0, 8] + out[1, 8]
    return tot / jnp.maximum(cnt, jnp.float32(1.0))


# speedup vs baseline: 1.0201x; 1.0201x over previous
import functools
import numpy as np
import jax
import jax.numpy as jnp
from jax import lax
from jax.experimental import pallas as pl
from jax.experimental.pallas import tpu as pltpu
from jax.experimental.pallas import tpu_sc as plsc

_IGNORE = 255
_MIN_KEPT = 65536
_B, _C, _H, _W = 4, 19, 512, 512
_N = _B * _H * _W
_ROWS = _H * _W // 128
_RBLK = 64

_NT = 16
_CHUNK = _N // _NT
_NV = _CHUNK // 16
_BINS = 256
_NREG = 2
_INT_MIN = np.int32(-2147483648)


def _key_i32(s):
    b = lax.bitcast_convert_type(s, jnp.int32)
    return b ^ (lax.shift_right_arithmetic(b, 31) | _INT_MIN)


def _thresh_const():
    b = np.float32(np.log(np.float32(0.6))).view(np.int32)
    u = np.uint32(b) ^ (np.uint32(np.int32(b) >> np.int32(31))
                        | np.uint32(0x80000000))
    return int(np.int32(u ^ np.uint32(0x80000000)))

_T06_SIGNED = _thresh_const()


def _tc_body(pred_ref, tgt_ref, key_ref):
    x = pred_ref[0]
    t = tgt_ref[0]
    m = jnp.max(x, axis=0)
    lse = m + jnp.log(jnp.sum(jnp.exp(x - m[None]), axis=0))
    cls = lax.broadcasted_iota(jnp.int32, x.shape, 0)
    xt = jnp.max(jnp.where(cls == t[None], x, -jnp.inf), axis=0)
    s = jnp.where(t != _IGNORE, xt - lse, jnp.float32(1e30))
    key_ref[0] = _key_i32(s).reshape(_RBLK * 4, 128)


_tc_keys = pl.pallas_call(
    _tc_body,
    grid=(_B, _H // _RBLK),
    in_specs=[
        pl.BlockSpec((1, _C, _RBLK, _W), lambda b, i: (b, 0, i, 0)),
        pl.BlockSpec((1, _RBLK, _W), lambda b, i: (b, i, 0)),
    ],
    out_specs=pl.BlockSpec((1, _RBLK * 4, 128), lambda b, i: (b, i, 0)),
    out_shape=jax.ShapeDtypeStruct((_B, _ROWS, 128), jnp.int32),
)


def _sc_body(keys_hbm, out_hbm, keys_v, hist_v, hist2_v, histr_v, ctrl_v,
             red_v, outst_v, hist_sh, ctrl_sh, sums_sh, cnts_sh):
    c = lax.axis_index("c")
    sid = lax.axis_index("s")
    lane = lax.iota(jnp.int32, 16)
    lane_base = lane * _BINS
    ones = jnp.ones((16,), jnp.int32)

    pltpu.sync_copy(keys_hbm.at[pl.ds(sid * _CHUNK, _CHUNK)], keys_v)

    @plsc.parallel_loop(0, _NREG * _NT * _BINS // 16, unroll=8)
    def _(i):
        hist_v[pl.ds(i * 16, 16)] = jnp.zeros((16,), jnp.int32)

    prefix = jnp.zeros((16,), jnp.int32)
    krem = jnp.full((16,), _MIN_KEPT, jnp.int32)

    for p in range(4):
        shift = jnp.full((16,), 24 - 8 * p, jnp.int32)
        hshift = jnp.full((16,), 32 - 8 * p, jnp.int32)
        mask8 = jnp.full((16,), 0xFF, jnp.int32)

        @plsc.parallel_loop(0, _NV // 2, unroll=4)
        def _(i, prefix=prefix):
            for r in range(2):
                v = keys_v[pl.ds((2 * i + r) * 16, 16)]
                idx = lane_base + (lax.shift_right_logical(v, shift) & mask8)
                idx = idx + r * (_NT * _BINS)
                if p == 0:
                    plsc.addupdate_scatter(hist_v, [idx], ones)
                else:
                    cand = lax.shift_right_logical(v, hshift) == prefix
                    plsc.addupdate_scatter(hist_v, [idx], ones, mask=cand)

        @plsc.parallel_loop(0, _BINS // 16, unroll=2)
        def _(j):
            acc = hist_v[pl.ds(j * 16, 16)]
            hist_v[pl.ds(j * 16, 16)] = jnp.zeros((16,), jnp.int32)
            for l in range(1, _NREG * _NT):
                acc = acc + hist_v[pl.ds(l * _BINS + j * 16, 16)]
                hist_v[pl.ds(l * _BINS + j * 16, 16)] = jnp.zeros((16,), jnp.int32)
            histr_v[pl.ds(j * 16, 16)] = acc

        pltpu.sync_copy(histr_v, hist_sh.at[pl.ds(sid * _BINS, _BINS)])
        plsc.subcore_barrier()

        @pl.when(sid == 0)
        def _():
            pltpu.sync_copy(hist_sh, hist2_v)
            carry = jnp.zeros((16,), jnp.int32)
            dsel = jnp.zeros((16,), jnp.int32)
            below = jnp.zeros((16,), jnp.int32)
            for j in range(_BINS // 16):
                tot = hist2_v[pl.ds(j * 16, 16)]
                for tt in range(1, _NT):
                    tot = tot + hist2_v[pl.ds(tt * _BINS + j * 16, 16)]
                cum = plsc.cumsum(tot) + carry
                prev = cum - tot
                isd = (cum >= krem) & (prev < krem)
                bid = jnp.full((16,), j * 16, jnp.int32) + lane
                dsel = dsel + jnp.where(isd, bid, 0)
                below = below + jnp.where(isd, prev, 0)
                carry = carry + jnp.broadcast_to(jnp.sum(tot), (16,))
            dvec = jnp.broadcast_to(jnp.sum(dsel), (16,))
            bvec = jnp.broadcast_to(jnp.sum(below), (16,))
            eight = jnp.full((16,), 8, jnp.int32)
            ctrl_v[pl.ds(0, 16)] = lax.shift_left(prefix, eight) | dvec
            ctrl_v[pl.ds(16, 16)] = krem - bvec
            pltpu.sync_copy(ctrl_v, ctrl_sh)

        plsc.subcore_barrier()
        pltpu.sync_copy(ctrl_sh, ctrl_v)
        prefix = ctrl_v[pl.ds(0, 16)]
        krem = ctrl_v[pl.ds(16, 16)]

    tsig = jnp.maximum(prefix ^ _INT_MIN,
                       jnp.full((16,), _T06_SIGNED, jnp.int32))
    start = c * (_NV // 2)

    zf = jnp.zeros((16,), jnp.float32)

    @plsc.parallel_loop(0, _NV // 4, unroll=4, carry=(zf, zf, zf, zf))
    def facc(i, acc):
        a0, c0, a1, c1 = acc
        for r in range(2):
            v = keys_v[pl.ds((start + 2 * i + r) * 16, 16)]
            kept = (v ^ _INT_MIN) <= tsig
            nv = v ^ jnp.full((16,), -1, jnp.int32)
            b = v ^ (lax.shift_right_arithmetic(nv, 31) | _INT_MIN)
            s = plsc.bitcast(b, jnp.float32)
            ds = jnp.where(kept, -s, jnp.float32(0.0))
            dc = jnp.where(kept, jnp.float32(1.0), jnp.float32(0.0))
            if r == 0:
                a0, c0 = a0 + ds, c0 + dc
            else:
                a1, c1 = a1 + ds, c1 + dc
        return (a0, c0, a1, c1)

    asum = facc[0] + facc[2]
    acnt = facc[1] + facc[3]

    outst_v[...] = asum
    pltpu.sync_copy(outst_v, sums_sh.at[pl.ds(sid * 16, 16)])
    outst_v[...] = acnt
    pltpu.sync_copy(outst_v, cnts_sh.at[pl.ds(sid * 16, 16)])
    plsc.subcore_barrier()

    @pl.when(sid == 0)
    def _():
        pltpu.sync_copy(sums_sh, red_v)
        stot = red_v[pl.ds(0, 16)]
        for tt in range(1, _NT):
            stot = stot + red_v[pl.ds(tt * 16, 16)]
        ssum = jnp.broadcast_to(jnp.sum(stot), (16,))
        pltpu.sync_copy(cnts_sh, red_v)
        ctot = red_v[pl.ds(0, 16)]
        for tt in range(1, _NT):
            ctot = ctot + red_v[pl.ds(tt * 16, 16)]
        csum = jnp.broadcast_to(jnp.sum(ctot), (16,))
        outst_v[...] = jnp.where(lane < 8, ssum, csum)
        pltpu.sync_copy(outst_v, out_hbm.at[c])


@functools.cache
def _get_sc_select():
    return functools.partial(
        pl.kernel,
        out_type=jax.ShapeDtypeStruct((2, 16), jnp.float32),
        mesh=plsc.VectorSubcoreMesh(core_axis_name="c", subcore_axis_name="s"),
        compiler_params=pltpu.CompilerParams(needs_layout_passes=False),
        scratch_types=[
            pltpu.VMEM((_CHUNK,), jnp.int32),
            pltpu.VMEM((_NREG * _NT * _BINS,), jnp.int32),
            pltpu.VMEM((_NT * _BINS,), jnp.int32),
            pltpu.VMEM((_BINS,), jnp.int32),
            pltpu.VMEM((32,), jnp.int32),
            pltpu.VMEM((_NT * 16,), jnp.float32),
            pltpu.VMEM((16,), jnp.float32),
            pltpu.VMEM_SHARED((_NT * _BINS,), jnp.int32),
            pltpu.VMEM_SHARED((32,), jnp.int32),
            pltpu.VMEM_SHARED((_NT * 16,), jnp.float32),
            pltpu.VMEM_SHARED((_NT * 16,), jnp.float32),
        ],
    )(_sc_body)


def kernel(pred, target):
    keys = _tc_keys(pred, target.astype(jnp.int32)).reshape(-1)
    out = _get_sc_select()(keys)
    tot = out[0, 0] + out[1, 0]
    cnt = out[0, 8] + out[1, 8]
    return tot / jnp.maximum(cnt, jnp.float32(1.0))


# speedup vs baseline: 1.1548x; 1.1321x over previous
import functools
import numpy as np
import jax
import jax.numpy as jnp
from jax import lax
from jax.experimental import pallas as pl
from jax.experimental.pallas import tpu as pltpu
from jax.experimental.pallas import tpu_sc as plsc

_IGNORE = 255
_MIN_KEPT = 65536
_B, _C, _H, _W = 4, 19, 512, 512
_N = _B * _H * _W
_ROWS = _H * _W // 128
_RBLK = 64

_NT = 16
_CHUNK = _N // _NT
_NV = _CHUNK // 16
_BINS = 256
_NREG = 2
_INT_MIN = np.int32(-2147483648)


def _key_i32(s):
    b = lax.bitcast_convert_type(s, jnp.int32)
    return b ^ (lax.shift_right_arithmetic(b, 31) | _INT_MIN)


def _thresh_const():
    b = np.float32(np.log(np.float32(0.6))).view(np.int32)
    u = np.uint32(b) ^ (np.uint32(np.int32(b) >> np.int32(31))
                        | np.uint32(0x80000000))
    return int(np.int32(u ^ np.uint32(0x80000000)))

_T06_SIGNED = _thresh_const()


def _tc_body(pred_ref, tgt_ref, key_ref):
    x = pred_ref[0]
    t = tgt_ref[0]
    m = jnp.max(x, axis=0)
    lse = m + jnp.log(jnp.sum(jnp.exp(x - m[None]), axis=0))
    cls = lax.broadcasted_iota(jnp.int32, x.shape, 0)
    xt = jnp.max(jnp.where(cls == t[None], x, -jnp.inf), axis=0)
    s = jnp.where(t != _IGNORE, xt - lse, jnp.float32(1e30))
    key_ref[0] = _key_i32(s).reshape(_RBLK * 4, 128)


_tc_keys = pl.pallas_call(
    _tc_body,
    grid=(_B, _H // _RBLK),
    in_specs=[
        pl.BlockSpec((1, _C, _RBLK, _W), lambda b, i: (b, 0, i, 0)),
        pl.BlockSpec((1, _RBLK, _W), lambda b, i: (b, i, 0)),
    ],
    out_specs=pl.BlockSpec((1, _RBLK * 4, 128), lambda b, i: (b, i, 0)),
    out_shape=jax.ShapeDtypeStruct((_B, _ROWS, 128), jnp.int32),
)


def _sc_body(keys_hbm, out_hbm, keys_v, hist_v, hist2_v, histr_v, ctrl_v,
             red_v, outst_v, hist_sh, ctrl_sh, sums_sh, cnts_sh):
    c = lax.axis_index("c")
    sid = lax.axis_index("s")
    lane = lax.iota(jnp.int32, 16)
    lane_base = lane * _BINS
    ones = jnp.ones((16,), jnp.int32)

    pltpu.sync_copy(keys_hbm.at[pl.ds(sid * _CHUNK, _CHUNK)], keys_v)

    @plsc.parallel_loop(0, _NREG * _NT * _BINS // 16, unroll=8)
    def _(i):
        hist_v[pl.ds(i * 16, 16)] = jnp.zeros((16,), jnp.int32)

    prefix = jnp.zeros((16,), jnp.int32)
    krem = jnp.full((16,), _MIN_KEPT, jnp.int32)

    for p in range(4):
        shift = jnp.full((16,), 24 - 8 * p, jnp.int32)
        hshift = jnp.full((16,), 32 - 8 * p, jnp.int32)
        mask8 = jnp.full((16,), 0xFF, jnp.int32)

        if p == 0:
            @plsc.parallel_loop(0, _NV // 2, unroll=4)
            def _(i):
                v0 = keys_v[pl.ds((2 * i) * 16, 16)]
                v1 = keys_v[pl.ds((2 * i + 1) * 16, 16)]
                d0 = lane_base + (lax.shift_right_logical(v0, shift) & mask8)
                d1 = lane_base + (lax.shift_right_logical(v1, shift) & mask8)
                eqm = d0 == d1
                val = jnp.where(eqm, jnp.full((16,), 2, jnp.int32), ones)
                plsc.addupdate_scatter(hist_v, [d0], val)
                plsc.addupdate_scatter(hist_v, [d1 + (_NT * _BINS)], ones,
                                       mask=~eqm)
        else:
            @plsc.parallel_loop(0, _NV // 2, unroll=4)
            def _(i, prefix=prefix):
                for r in range(2):
                    v = keys_v[pl.ds((2 * i + r) * 16, 16)]
                    idx = lane_base + (lax.shift_right_logical(v, shift) & mask8)
                    idx = idx + r * (_NT * _BINS)
                    cand = lax.shift_right_logical(v, hshift) == prefix
                    plsc.addupdate_scatter(hist_v, [idx], ones, mask=cand)

        @plsc.parallel_loop(0, _BINS // 16, unroll=2)
        def _(j):
            acc = hist_v[pl.ds(j * 16, 16)]
            hist_v[pl.ds(j * 16, 16)] = jnp.zeros((16,), jnp.int32)
            for l in range(1, _NREG * _NT):
                acc = acc + hist_v[pl.ds(l * _BINS + j * 16, 16)]
                hist_v[pl.ds(l * _BINS + j * 16, 16)] = jnp.zeros((16,), jnp.int32)
            histr_v[pl.ds(j * 16, 16)] = acc

        pltpu.sync_copy(histr_v, hist_sh.at[pl.ds(sid * _BINS, _BINS)])
        plsc.subcore_barrier()

        @pl.when(sid == 0)
        def _():
            pltpu.sync_copy(hist_sh, hist2_v)
            carry = jnp.zeros((16,), jnp.int32)
            dsel = jnp.zeros((16,), jnp.int32)
            below = jnp.zeros((16,), jnp.int32)
            for j in range(_BINS // 16):
                tot = hist2_v[pl.ds(j * 16, 16)]
                for tt in range(1, _NT):
                    tot = tot + hist2_v[pl.ds(tt * _BINS + j * 16, 16)]
                cum = plsc.cumsum(tot) + carry
                prev = cum - tot
                isd = (cum >= krem) & (prev < krem)
                bid = jnp.full((16,), j * 16, jnp.int32) + lane
                dsel = dsel + jnp.where(isd, bid, 0)
                below = below + jnp.where(isd, prev, 0)
                carry = carry + jnp.broadcast_to(jnp.sum(tot), (16,))
            dvec = jnp.broadcast_to(jnp.sum(dsel), (16,))
            bvec = jnp.broadcast_to(jnp.sum(below), (16,))
            eight = jnp.full((16,), 8, jnp.int32)
            ctrl_v[pl.ds(0, 16)] = lax.shift_left(prefix, eight) | dvec
            ctrl_v[pl.ds(16, 16)] = krem - bvec
            pltpu.sync_copy(ctrl_v, ctrl_sh)

        plsc.subcore_barrier()
        pltpu.sync_copy(ctrl_sh, ctrl_v)
        prefix = ctrl_v[pl.ds(0, 16)]
        krem = ctrl_v[pl.ds(16, 16)]

    tsig = jnp.maximum(prefix ^ _INT_MIN,
                       jnp.full((16,), _T06_SIGNED, jnp.int32))
    start = c * (_NV // 2)

    zf = jnp.zeros((16,), jnp.float32)

    @plsc.parallel_loop(0, _NV // 4, unroll=4, carry=(zf, zf, zf, zf))
    def facc(i, acc):
        a0, c0, a1, c1 = acc
        for r in range(2):
            v = keys_v[pl.ds((start + 2 * i + r) * 16, 16)]
            kept = (v ^ _INT_MIN) <= tsig
            nv = v ^ jnp.full((16,), -1, jnp.int32)
            b = v ^ (lax.shift_right_arithmetic(nv, 31) | _INT_MIN)
            s = plsc.bitcast(b, jnp.float32)
            ds = jnp.where(kept, -s, jnp.float32(0.0))
            dc = jnp.where(kept, jnp.float32(1.0), jnp.float32(0.0))
            if r == 0:
                a0, c0 = a0 + ds, c0 + dc
            else:
                a1, c1 = a1 + ds, c1 + dc
        return (a0, c0, a1, c1)

    asum = facc[0] + facc[2]
    acnt = facc[1] + facc[3]

    outst_v[...] = asum
    pltpu.sync_copy(outst_v, sums_sh.at[pl.ds(sid * 16, 16)])
    outst_v[...] = acnt
    pltpu.sync_copy(outst_v, cnts_sh.at[pl.ds(sid * 16, 16)])
    plsc.subcore_barrier()

    @pl.when(sid == 0)
    def _():
        pltpu.sync_copy(sums_sh, red_v)
        stot = red_v[pl.ds(0, 16)]
        for tt in range(1, _NT):
            stot = stot + red_v[pl.ds(tt * 16, 16)]
        ssum = jnp.broadcast_to(jnp.sum(stot), (16,))
        pltpu.sync_copy(cnts_sh, red_v)
        ctot = red_v[pl.ds(0, 16)]
        for tt in range(1, _NT):
            ctot = ctot + red_v[pl.ds(tt * 16, 16)]
        csum = jnp.broadcast_to(jnp.sum(ctot), (16,))
        outst_v[...] = jnp.where(lane < 8, ssum, csum)
        pltpu.sync_copy(outst_v, out_hbm.at[c])


@functools.cache
def _get_sc_select():
    return functools.partial(
        pl.kernel,
        out_type=jax.ShapeDtypeStruct((2, 16), jnp.float32),
        mesh=plsc.VectorSubcoreMesh(core_axis_name="c", subcore_axis_name="s"),
        compiler_params=pltpu.CompilerParams(needs_layout_passes=False),
        scratch_types=[
            pltpu.VMEM((_CHUNK,), jnp.int32),
            pltpu.VMEM((_NREG * _NT * _BINS,), jnp.int32),
            pltpu.VMEM((_NT * _BINS,), jnp.int32),
            pltpu.VMEM((_BINS,), jnp.int32),
            pltpu.VMEM((32,), jnp.int32),
            pltpu.VMEM((_NT * 16,), jnp.float32),
            pltpu.VMEM((16,), jnp.float32),
            pltpu.VMEM_SHARED((_NT * _BINS,), jnp.int32),
            pltpu.VMEM_SHARED((32,), jnp.int32),
            pltpu.VMEM_SHARED((_NT * 16,), jnp.float32),
            pltpu.VMEM_SHARED((_NT * 16,), jnp.float32),
        ],
    )(_sc_body)


def kernel(pred, target):
    keys = _tc_keys(pred, target.astype(jnp.int32)).reshape(-1)
    out = _get_sc_select()(keys)
    tot = out[0, 0] + out[1, 0]
    cnt = out[0, 8] + out[1, 8]
    return tot / jnp.maximum(cnt, jnp.float32(1.0))


# speedup vs baseline: 1.2319x; 1.0667x over previous
import functools
import numpy as np
import jax
import jax.numpy as jnp
from jax import lax
from jax.experimental import pallas as pl
from jax.experimental.pallas import tpu as pltpu
from jax.experimental.pallas import tpu_sc as plsc

_IGNORE = 255
_MIN_KEPT = 65536
_B, _C, _H, _W = 4, 19, 512, 512
_N = _B * _H * _W
_ROWS = _H * _W // 128
_RBLK = 64

_NT = 16
_CHUNK = _N // _NT
_NV = _CHUNK // 16
_BINS = 256
_NREG = 2
_INT_MIN = np.int32(-2147483648)


def _key_i32(s):
    b = lax.bitcast_convert_type(s, jnp.int32)
    return b ^ (lax.shift_right_arithmetic(b, 31) | _INT_MIN)


def _thresh_const():
    b = np.float32(np.log(np.float32(0.6))).view(np.int32)
    u = np.uint32(b) ^ (np.uint32(np.int32(b) >> np.int32(31))
                        | np.uint32(0x80000000))
    return int(np.int32(u ^ np.uint32(0x80000000)))

_T06_SIGNED = _thresh_const()


def _tc_body(pred_ref, tgt_ref, key_ref):
    x = pred_ref[0]
    t = tgt_ref[0]
    m = jnp.max(x, axis=0)
    lse = m + jnp.log(jnp.sum(jnp.exp(x - m[None]), axis=0))
    cls = lax.broadcasted_iota(jnp.int32, x.shape, 0)
    xt = jnp.max(jnp.where(cls == t[None], x, -jnp.inf), axis=0)
    s = jnp.where(t != _IGNORE, xt - lse, jnp.float32(1e30))
    key_ref[0] = _key_i32(s).reshape(_RBLK * 4, 128)


_tc_keys = pl.pallas_call(
    _tc_body,
    grid=(_B, _H // _RBLK),
    in_specs=[
        pl.BlockSpec((1, _C, _RBLK, _W), lambda b, i: (b, 0, i, 0)),
        pl.BlockSpec((1, _RBLK, _W), lambda b, i: (b, i, 0)),
    ],
    out_specs=pl.BlockSpec((1, _RBLK * 4, 128), lambda b, i: (b, i, 0)),
    out_shape=jax.ShapeDtypeStruct((_B, _ROWS, 128), jnp.int32),
)


def _sc_body(keys_hbm, out_hbm, keys_v, hist_v, hist2_v, histr_v, ctrl_v,
             red_v, outst_v, hist_sh, ctrl_sh, sums_sh, cnts_sh):
    c = lax.axis_index("c")
    sid = lax.axis_index("s")
    lane = lax.iota(jnp.int32, 16)
    lane_base = lane * _BINS
    ones = jnp.ones((16,), jnp.int32)

    pltpu.sync_copy(keys_hbm.at[pl.ds(sid * _CHUNK, _CHUNK)], keys_v)

    @plsc.parallel_loop(0, _NREG * _NT * _BINS // 16, unroll=8)
    def _(i):
        hist_v[pl.ds(i * 16, 16)] = jnp.zeros((16,), jnp.int32)

    prefix = jnp.zeros((16,), jnp.int32)
    krem = jnp.full((16,), _MIN_KEPT, jnp.int32)

    for p in range(4):
        shift = jnp.full((16,), 24 - 8 * p, jnp.int32)
        hshift = jnp.full((16,), 32 - 8 * p, jnp.int32)
        mask8 = jnp.full((16,), 0xFF, jnp.int32)

        if p == 0:
            @plsc.parallel_loop(0, _NV // 4, unroll=2)
            def _(i):
                v0 = keys_v[pl.ds((4 * i) * 16, 16)]
                v1 = keys_v[pl.ds((4 * i + 1) * 16, 16)]
                v2 = keys_v[pl.ds((4 * i + 2) * 16, 16)]
                v3 = keys_v[pl.ds((4 * i + 3) * 16, 16)]
                d0 = lane_base + (lax.shift_right_logical(v0, shift) & mask8)
                d1 = lane_base + (lax.shift_right_logical(v1, shift) & mask8)
                d2 = lane_base + (lax.shift_right_logical(v2, shift) & mask8)
                d3 = lane_base + (lax.shift_right_logical(v3, shift) & mask8)
                two = jnp.full((16,), 2, jnp.int32)
                eq01 = d0 == d1
                eq23 = d2 == d3
                va = jnp.where(eq01, two, ones)
                vb = jnp.where(eq23, two, ones)
                eqab = d0 == d2
                plsc.addupdate_scatter(hist_v, [d0],
                                       jnp.where(eqab, va + vb, va))
                plsc.addupdate_scatter(hist_v, [d2 + (_NT * _BINS)], vb,
                                       mask=~eqab)
                plsc.addupdate_scatter(hist_v, [d1], ones, mask=~eq01)
                plsc.addupdate_scatter(hist_v, [d3 + (_NT * _BINS)], ones,
                                       mask=~eq23)
        else:
            @plsc.parallel_loop(0, _NV // 2, unroll=4)
            def _(i, prefix=prefix):
                for r in range(2):
                    v = keys_v[pl.ds((2 * i + r) * 16, 16)]
                    idx = lane_base + (lax.shift_right_logical(v, shift) & mask8)
                    idx = idx + r * (_NT * _BINS)
                    cand = lax.shift_right_logical(v, hshift) == prefix
                    plsc.addupdate_scatter(hist_v, [idx], ones, mask=cand)

        @plsc.parallel_loop(0, _BINS // 16, unroll=2)
        def _(j):
            acc = hist_v[pl.ds(j * 16, 16)]
            hist_v[pl.ds(j * 16, 16)] = jnp.zeros((16,), jnp.int32)
            for l in range(1, _NREG * _NT):
                acc = acc + hist_v[pl.ds(l * _BINS + j * 16, 16)]
                hist_v[pl.ds(l * _BINS + j * 16, 16)] = jnp.zeros((16,), jnp.int32)
            histr_v[pl.ds(j * 16, 16)] = acc

        pltpu.sync_copy(histr_v, hist_sh.at[pl.ds(sid * _BINS, _BINS)])
        plsc.subcore_barrier()

        @pl.when(sid == 0)
        def _():
            pltpu.sync_copy(hist_sh, hist2_v)
            carry = jnp.zeros((16,), jnp.int32)
            dsel = jnp.zeros((16,), jnp.int32)
            below = jnp.zeros((16,), jnp.int32)
            for j in range(_BINS // 16):
                tot = hist2_v[pl.ds(j * 16, 16)]
                for tt in range(1, _NT):
                    tot = tot + hist2_v[pl.ds(tt * _BINS + j * 16, 16)]
                cum = plsc.cumsum(tot) + carry
                prev = cum - tot
                isd = (cum >= krem) & (prev < krem)
                bid = jnp.full((16,), j * 16, jnp.int32) + lane
                dsel = dsel + jnp.where(isd, bid, 0)
                below = below + jnp.where(isd, prev, 0)
                carry = carry + jnp.broadcast_to(jnp.sum(tot), (16,))
            dvec = jnp.broadcast_to(jnp.sum(dsel), (16,))
            bvec = jnp.broadcast_to(jnp.sum(below), (16,))
            eight = jnp.full((16,), 8, jnp.int32)
            ctrl_v[pl.ds(0, 16)] = lax.shift_left(prefix, eight) | dvec
            ctrl_v[pl.ds(16, 16)] = krem - bvec
            pltpu.sync_copy(ctrl_v, ctrl_sh)

        plsc.subcore_barrier()
        pltpu.sync_copy(ctrl_sh, ctrl_v)
        prefix = ctrl_v[pl.ds(0, 16)]
        krem = ctrl_v[pl.ds(16, 16)]

    tsig = jnp.maximum(prefix ^ _INT_MIN,
                       jnp.full((16,), _T06_SIGNED, jnp.int32))
    start = c * (_NV // 2)

    zf = jnp.zeros((16,), jnp.float32)

    @plsc.parallel_loop(0, _NV // 4, unroll=4, carry=(zf, zf, zf, zf))
    def facc(i, acc):
        a0, c0, a1, c1 = acc
        for r in range(2):
            v = keys_v[pl.ds((start + 2 * i + r) * 16, 16)]
            kept = (v ^ _INT_MIN) <= tsig
            nv = v ^ jnp.full((16,), -1, jnp.int32)
            b = v ^ (lax.shift_right_arithmetic(nv, 31) | _INT_MIN)
            s = plsc.bitcast(b, jnp.float32)
            ds = jnp.where(kept, -s, jnp.float32(0.0))
            dc = jnp.where(kept, jnp.float32(1.0), jnp.float32(0.0))
            if r == 0:
                a0, c0 = a0 + ds, c0 + dc
            else:
                a1, c1 = a1 + ds, c1 + dc
        return (a0, c0, a1, c1)

    asum = facc[0] + facc[2]
    acnt = facc[1] + facc[3]

    outst_v[...] = asum
    pltpu.sync_copy(outst_v, sums_sh.at[pl.ds(sid * 16, 16)])
    outst_v[...] = acnt
    pltpu.sync_copy(outst_v, cnts_sh.at[pl.ds(sid * 16, 16)])
    plsc.subcore_barrier()

    @pl.when(sid == 0)
    def _():
        pltpu.sync_copy(sums_sh, red_v)
        stot = red_v[pl.ds(0, 16)]
        for tt in range(1, _NT):
            stot = stot + red_v[pl.ds(tt * 16, 16)]
        ssum = jnp.broadcast_to(jnp.sum(stot), (16,))
        pltpu.sync_copy(cnts_sh, red_v)
        ctot = red_v[pl.ds(0, 16)]
        for tt in range(1, _NT):
            ctot = ctot + red_v[pl.ds(tt * 16, 16)]
        csum = jnp.broadcast_to(jnp.sum(ctot), (16,))
        outst_v[...] = jnp.where(lane < 8, ssum, csum)
        pltpu.sync_copy(outst_v, out_hbm.at[c])


@functools.cache
def _get_sc_select():
    return functools.partial(
        pl.kernel,
        out_type=jax.ShapeDtypeStruct((2, 16), jnp.float32),
        mesh=plsc.VectorSubcoreMesh(core_axis_name="c", subcore_axis_name="s"),
        compiler_params=pltpu.CompilerParams(needs_layout_passes=False),
        scratch_types=[
            pltpu.VMEM((_CHUNK,), jnp.int32),
            pltpu.VMEM((_NREG * _NT * _BINS,), jnp.int32),
            pltpu.VMEM((_NT * _BINS,), jnp.int32),
            pltpu.VMEM((_BINS,), jnp.int32),
            pltpu.VMEM((32,), jnp.int32),
            pltpu.VMEM((_NT * 16,), jnp.float32),
            pltpu.VMEM((16,), jnp.float32),
            pltpu.VMEM_SHARED((_NT * _BINS,), jnp.int32),
            pltpu.VMEM_SHARED((32,), jnp.int32),
            pltpu.VMEM_SHARED((_NT * 16,), jnp.float32),
            pltpu.VMEM_SHARED((_NT * 16,), jnp.float32),
        ],
    )(_sc_body)


def kernel(pred, target):
    keys = _tc_keys(pred, target.astype(jnp.int32)).reshape(-1)
    out = _get_sc_select()(keys)
    tot = out[0, 0] + out[1, 0]
    cnt = out[0, 8] + out[1, 8]
    return tot / jnp.maximum(cnt, jnp.float32(1.0))
